# baseline (speedup 1.0000x reference)
import functools

import jax
import jax.numpy as jnp
from jax import lax
from jax.experimental import pallas as pl
from jax.experimental.pallas import tpu as pltpu
from jax.experimental.pallas import tpu_sc as plsc

_NUM_CORES = 2
_NUM_SUBCORES = 16
_NUM_WORKERS = _NUM_CORES * _NUM_SUBCORES

_CH = 128
_L = 16
_NBUF = 4
_WBUF = 2


@jax.jit
def _embedding_lookup(idx_t, table_r):
    seq, batch = idx_t.shape
    vhalf, two_d = table_r.shape
    d = two_d // 2
    mesh = plsc.VectorSubcoreMesh(core_axis_name="c", subcore_axis_name="s")

    @functools.partial(
        pl.kernel,
        out_type=jax.ShapeDtypeStruct((seq, d, batch), jnp.float32),
        mesh=mesh,
        scratch_types=[
            pltpu.VMEM((seq, _CH), jnp.int32),
            pltpu.VMEM((_NBUF, _CH), jnp.int32),
            pltpu.VMEM((_NBUF, _CH, two_d), jnp.float32),
            pltpu.VMEM((_WBUF, d, _CH), jnp.float32),
            pltpu.SemaphoreType.DMA((_NBUF,)),
            pltpu.SemaphoreType.DMA((_WBUF,)),
        ],
        compiler_params=pltpu.CompilerParams(
            use_tc_tiling_on_sc=True, needs_layout_passes=False
        ),
    )
    def emb(idx_hbm, table_hbm, out_hbm, idx_v, h_v, rows_v, tr_v, gsem, wsem):
        wid = lax.axis_index("s") * _NUM_CORES + lax.axis_index("c")
        b0 = wid * _CH

        pltpu.sync_copy(idx_hbm.at[:, pl.ds(b0, _CH)], idx_v)

        iotas = [lax.iota(jnp.int32, _L) + j0 for j0 in range(0, _CH, _L)]

        def start_gather(s, b):
            for k in range(_CH // _L):
                sl = pl.ds(k * _L, _L)
                h_v[b, sl] = jax.lax.shift_right_logical(idx_v[s, sl], 1)
            pltpu.async_copy(table_hbm.at[h_v.at[b]], rows_v.at[b], gsem.at[b])

        def wait_gather(b):
            pltpu.make_async_copy(
                table_hbm.at[pl.ds(0, _CH)], rows_v.at[b], gsem.at[b]
            ).wait()

        def start_write(t, s):
            pltpu.async_copy(
                tr_v.at[t], out_hbm.at[s, :, pl.ds(b0, _CH)], wsem.at[t]
            )

        def wait_write(t):
            pltpu.make_async_copy(
                out_hbm.at[0, :, pl.ds(b0, _CH)], tr_v.at[t], wsem.at[t]
            ).wait()

        def transpose_block(s, b, t):
            src = rows_v.at[b]
            dst = tr_v.at[t]
            offs = [
                jax.lax.shift_left(
                    jnp.bitwise_and(idx_v[s, pl.ds(k * _L, _L)], 1), 6
                )
                for k in range(_CH // _L)
            ]

            @pl.loop(0, d)
            def _(dd):
                col = jnp.full((_L,), dd, jnp.int32)
                for k in range(_CH // _L):
                    vals = plsc.load_gather(src, [iotas[k], offs[k] + col])
                    dst[dd, pl.ds(k * _L, _L)] = vals

        for b in range(_NBUF):
            start_gather(b, b)

        n_grp = seq // _NBUF

        @pl.loop(0, n_grp)
        def _(grp):
            s0 = grp * _NBUF
            for b in range(_NBUF):
                t = b % _WBUF
                wait_gather(b)

                @pl.when(jnp.logical_or(grp > 0, b >= _WBUF))
                def _():
                    wait_write(t)

                transpose_block(s0 + b, b, t)
                start_write(t, s0 + b)

                @pl.when(grp < n_grp - 1)
                def _():
                    start_gather(s0 + b + _NBUF, b)

        for t in range(_WBUF):
            wait_write(t)

    return emb(idx_t, table_r)


def kernel(inputs, table):
    batch, seq = inputs.shape
    v, d = table.shape
    idx_t = inputs.T
    table_r = jnp.reshape(table, (v // 2, d * 2))
    out3 = _embedding_lookup(idx_t, table_r)
    return jnp.transpose(out3, (2, 0, 1))


# speedup vs baseline: 1.4587x; 1.4587x over previous
import functools

import jax
import jax.numpy as jnp
from jax import lax
from jax.experimental import pallas as pl
from jax.experimental.pallas import tpu as pltpu
from jax.experimental.pallas import tpu_sc as plsc

_NUM_CORES = 2
_NUM_SUBCORES = 16
_NUM_WORKERS = _NUM_CORES * _NUM_SUBCORES

_CH = 128
_L = 16
_NBUF = 4
_WBUF = 2


@jax.jit
def _embedding_lookup(idx_t, table_r):
    seq, batch = idx_t.shape
    vhalf, two_d = table_r.shape
    d = two_d // 2
    mesh = plsc.VectorSubcoreMesh(core_axis_name="c", subcore_axis_name="s")

    @functools.partial(
        pl.kernel,
        out_type=jax.ShapeDtypeStruct((seq, d, batch), jnp.float32),
        mesh=mesh,
        scratch_types=[
            pltpu.VMEM((seq, _CH), jnp.int32),
            pltpu.VMEM((_NBUF, _CH), jnp.int32),
            pltpu.VMEM((_NBUF, _CH, two_d), jnp.float32),
            pltpu.VMEM((_WBUF, d, _CH), jnp.float32),
            pltpu.SemaphoreType.DMA((_NBUF,)),
            pltpu.SemaphoreType.DMA((_WBUF,)),
        ],
        compiler_params=pltpu.CompilerParams(
            use_tc_tiling_on_sc=True, needs_layout_passes=False
        ),
    )
    def emb(idx_hbm, table_hbm, out_hbm, idx_v, h_v, rows_v, tr_v, gsem, wsem):
        wid = lax.axis_index("s") * _NUM_CORES + lax.axis_index("c")
        b0 = wid * _CH

        pltpu.sync_copy(idx_hbm.at[:, pl.ds(b0, _CH)], idx_v)

        iotas = [lax.iota(jnp.int32, _L) + j0 for j0 in range(0, _CH, _L)]

        def start_gather(s, b):
            for k in range(_CH // _L):
                sl = pl.ds(k * _L, _L)
                h_v[b, sl] = jax.lax.shift_right_logical(idx_v[s, sl], 1)
            pltpu.async_copy(table_hbm.at[h_v.at[b]], rows_v.at[b], gsem.at[b])

        def wait_gather(b):
            pltpu.make_async_copy(
                table_hbm.at[pl.ds(0, _CH)], rows_v.at[b], gsem.at[b]
            ).wait()

        def start_write(t, s):
            pltpu.async_copy(
                tr_v.at[t], out_hbm.at[s, :, pl.ds(b0, _CH)], wsem.at[t]
            )

        def wait_write(t):
            pltpu.make_async_copy(
                out_hbm.at[0, :, pl.ds(b0, _CH)], tr_v.at[t], wsem.at[t]
            ).wait()

        def transpose_block(s, b, t):
            src = rows_v.at[b]
            dst = tr_v.at[t]
            offs = [
                jax.lax.shift_left(
                    jnp.bitwise_and(idx_v[s, pl.ds(k * _L, _L)], 1), 6
                )
                for k in range(_CH // _L)
            ]

            @plsc.parallel_loop(0, d, unroll=4)
            def _(dd):
                col = jnp.full((_L,), dd, jnp.int32)
                for k in range(_CH // _L):
                    vals = plsc.load_gather(src, [iotas[k], offs[k] + col])
                    dst[dd, pl.ds(k * _L, _L)] = vals

        for b in range(_NBUF):
            start_gather(b, b)

        n_grp = seq // _NBUF

        @pl.loop(0, n_grp)
        def _(grp):
            s0 = grp * _NBUF
            for b in range(_NBUF):
                t = b % _WBUF
                wait_gather(b)

                @pl.when(jnp.logical_or(grp > 0, b >= _WBUF))
                def _():
                    wait_write(t)

                transpose_block(s0 + b, b, t)
                start_write(t, s0 + b)

                @pl.when(grp < n_grp - 1)
                def _():
                    start_gather(s0 + b + _NBUF, b)

        for t in range(_WBUF):
            wait_write(t)

    return emb(idx_t, table_r)


def kernel(inputs, table):
    batch, seq = inputs.shape
    v, d = table.shape
    idx_t = inputs.T
    table_r = jnp.reshape(table, (v // 2, d * 2))
    out3 = _embedding_lookup(idx_t, table_r)
    return jnp.transpose(out3, (2, 0, 1))
